# Initial kernel scaffold; baseline (speedup 1.0000x reference)
import functools

import jax
import jax.numpy as jnp
from jax import lax
from jax.experimental import pallas as pl
from jax.experimental.pallas import tpu as pltpu
from jax.experimental.pallas import tpu_sc as plsc

B = 16384
LX = 200
V = 1_000_000
D = 16
PACK = 128 // D
G = V // PACK

NC, NS, L = 2, 16, 16
NW = NC * NS
RPW = B // NW
C = 64
CW = C * LX
NCHUNK = RPW // C

_ROUND_MAGIC = jnp.float32(8388608.0)


def _stage1_body(e_ref, wd_ref, bb_ref, o_ref):
    o_ref[...] = (
        jnp.dot(e_ref[...], wd_ref[...], preferred_element_type=jnp.float32)
        + bb_ref[...]
    )


def _stage1(emb2, wd, bb):
    blk = 1000
    return pl.pallas_call(
        _stage1_body,
        grid=(G // blk,),
        in_specs=[
            pl.BlockSpec((blk, 128), lambda i: (i, 0)),
            pl.BlockSpec((128, PACK), lambda i: (0, 0)),
            pl.BlockSpec((1, PACK), lambda i: (0, 0)),
        ],
        out_specs=pl.BlockSpec((blk, PACK), lambda i: (i, 0)),
        out_shape=jax.ShapeDtypeStruct((G, PACK), jnp.float32),
    )(emb2, wd, bb)


_MESH = plsc.VectorSubcoreMesh(core_axis_name="c", subcore_axis_name="s")


@functools.partial(
    pl.kernel,
    mesh=_MESH,
    out_type=jax.ShapeDtypeStruct((B,), jnp.float32),
    scratch_types=[
        pltpu.VMEM((CW,), jnp.int32),
        pltpu.VMEM((CW,), jnp.float32),
        pltpu.VMEM((RPW,), jnp.float32),
        pltpu.SemaphoreType.DMA,
    ],
)
def _stage2(xf_hbm, s_hbm, out_hbm, idx_v, val_v, out_v, sem):
    wid = lax.axis_index("s") * NC + lax.axis_index("c")
    row0 = wid * RPW
    lanes = lax.iota(jnp.int32, (L,)) * LX

    def chunk_body(ci, carry):
        base = (row0 + ci * C) * LX
        pltpu.sync_copy(xf_hbm.at[pl.ds(base, CW)], idx_v)
        pltpu.async_copy(s_hbm.at[idx_v], val_v, sem).wait()
        for g in range(C // L):
            lanes_g = lanes + (g * L * LX)

            def sum_body(l, acc):
                return acc + plsc.load_gather(val_v, [lanes_g + l])

            acc = lax.fori_loop(
                0, LX, sum_body, jnp.zeros((L,), jnp.float32), unroll=8
            )
            y = jnp.float32(1.0) / (jnp.float32(1.0) + jnp.exp(-acc))
            y = y * jnp.float32(1e4)
            y = (y + _ROUND_MAGIC) - _ROUND_MAGIC
            y = y * jnp.float32(1e-4)
            out_v[pl.ds(ci * C + g * L, L)] = y
        return carry

    lax.fori_loop(0, NCHUNK, chunk_body, 0)
    pltpu.sync_copy(out_v, out_hbm.at[pl.ds(row0, RPW)])


def kernel(x, emb, W, b):
    w = W.reshape(D).astype(jnp.float32)
    wd = jnp.kron(jnp.eye(PACK, dtype=jnp.float32), (w / LX)[:, None])
    bb = jnp.full((1, PACK), b[0] / LX, dtype=jnp.float32)
    s = _stage1(emb.reshape(G, 128), wd, bb).reshape(V)
    xf = x.astype(jnp.int32).reshape(B * LX)
    out = _stage2(xf, s)
    return out.reshape(B, 1)


# speedup vs baseline: 7.4919x; 7.4919x over previous
import functools

import jax
import jax.numpy as jnp
from jax import lax
from jax.experimental import pallas as pl
from jax.experimental.pallas import tpu as pltpu
from jax.experimental.pallas import tpu_sc as plsc

B = 16384
LX = 200
V = 1_000_000
D = 16
PACK = 128 // D
G = V // PACK

NC, NS, L = 2, 16, 16
NW = NC * NS
RPW = B // NW
C = 32
CW = C * LX
NCHUNK = RPW // C

_ROUND_MAGIC = 8388608.0


def _stage1_body(e_ref, wd_ref, bb_ref, o_ref):
    o_ref[...] = (
        jnp.dot(e_ref[...], wd_ref[...], preferred_element_type=jnp.float32)
        + bb_ref[...]
    )


def _stage1(emb2, wd, bb):
    blk = 1000
    return pl.pallas_call(
        _stage1_body,
        grid=(G // blk,),
        in_specs=[
            pl.BlockSpec((blk, 128), lambda i: (i, 0)),
            pl.BlockSpec((128, PACK), lambda i: (0, 0)),
            pl.BlockSpec((1, PACK), lambda i: (0, 0)),
        ],
        out_specs=pl.BlockSpec((blk, PACK), lambda i: (i, 0)),
        out_shape=jax.ShapeDtypeStruct((G, PACK), jnp.float32),
    )(emb2, wd, bb)


_MESH = plsc.VectorSubcoreMesh(core_axis_name="c", subcore_axis_name="s")


@functools.partial(
    pl.kernel,
    mesh=_MESH,
    out_type=jax.ShapeDtypeStruct((B,), jnp.float32),
    scratch_types=[
        pltpu.VMEM((CW,), jnp.int32),
        pltpu.VMEM((CW,), jnp.float32),
        pltpu.VMEM((RPW,), jnp.float32),
        pltpu.SemaphoreType.DMA,
    ],
)
def _stage2(xp_hbm, s_hbm, out_hbm, idx_v, val_v, out_v, sem):
    wid = lax.axis_index("s") * NC + lax.axis_index("c")
    row0 = wid * RPW

    def chunk_body(ci, carry):
        base = (wid * NCHUNK + ci) * CW
        pltpu.sync_copy(xp_hbm.at[pl.ds(base, CW)], idx_v)
        pltpu.async_copy(s_hbm.at[idx_v], val_v, sem).wait()

        def sum_body(l, accs):
            return tuple(
                accs[g] + val_v[pl.ds(l * C + g * L, L)] for g in range(C // L)
            )

        accs = lax.fori_loop(
            0,
            LX,
            sum_body,
            tuple(jnp.zeros((L,), jnp.float32) for _ in range(C // L)),
            unroll=8,
        )
        for g in range(C // L):
            y = jnp.float32(1.0) / (jnp.float32(1.0) + jnp.exp(-accs[g]))
            y = y * jnp.float32(1e4)
            y = (y + jnp.float32(_ROUND_MAGIC)) - jnp.float32(_ROUND_MAGIC)
            y = y * jnp.float32(1e-4)
            out_v[pl.ds(ci * C + g * L, L)] = y
        return carry

    lax.fori_loop(0, NCHUNK, chunk_body, 0)
    pltpu.sync_copy(out_v, out_hbm.at[pl.ds(row0, RPW)])


def kernel(x, emb, W, b):
    w = W.reshape(D).astype(jnp.float32)
    wd = jnp.kron(jnp.eye(PACK, dtype=jnp.float32), (w / LX)[:, None])
    bb = jnp.full((1, PACK), b[0] / LX, dtype=jnp.float32)
    s = _stage1(emb.reshape(G, 128), wd, bb).reshape(V)
    xp = (
        x.astype(jnp.int32)
        .reshape(NW, NCHUNK, C, LX)
        .transpose(0, 1, 3, 2)
        .reshape(B * LX)
    )
    out = _stage2(xp, s)
    return out.reshape(B, 1)


# speedup vs baseline: 8.2148x; 1.0965x over previous
import functools

import jax
import jax.numpy as jnp
from jax import lax
from jax.experimental import pallas as pl
from jax.experimental.pallas import tpu as pltpu
from jax.experimental.pallas import tpu_sc as plsc

B = 16384
LX = 200
V = 1_000_000
D = 16
PACK = 128 // D
G = V // PACK

CBLK = 1024
NCOLP = 123 * CBLK
VP = PACK * NCOLP

NC, NS, L = 2, 16, 16
NW = NC * NS
RPW = B // NW
C = 128
CW = C * LX
NCHUNK = RPW // C

_ROUND_MAGIC = 8388608.0


def _stage1_body(e_ref, wd_ref, b_ref, o_ref):
    o_ref[...] = (
        lax.dot_general(
            wd_ref[...],
            e_ref[...],
            (((0,), (1,)), ((), ())),
            preferred_element_type=jnp.float32,
        )
        + b_ref[0, 0]
    )


def _stage1(emb2, wd, bscal):
    return pl.pallas_call(
        _stage1_body,
        grid=(NCOLP // CBLK,),
        in_specs=[
            pl.BlockSpec((CBLK, 128), lambda i: (i, 0)),
            pl.BlockSpec((128, PACK), lambda i: (0, 0)),
            pl.BlockSpec(memory_space=pltpu.SMEM),
        ],
        out_specs=pl.BlockSpec((PACK, CBLK), lambda i: (0, i)),
        out_shape=jax.ShapeDtypeStruct((PACK, NCOLP), jnp.float32),
    )(emb2, wd, bscal)


_MESH = plsc.VectorSubcoreMesh(core_axis_name="c", subcore_axis_name="s")


@functools.partial(
    pl.kernel,
    mesh=_MESH,
    out_type=jax.ShapeDtypeStruct((B,), jnp.float32),
    scratch_types=[
        pltpu.VMEM((CW,), jnp.int32),
        pltpu.VMEM((CW,), jnp.float32),
        pltpu.VMEM((RPW,), jnp.float32),
        pltpu.VMEM((2 * L,), jnp.float32),
        pltpu.SemaphoreType.DMA,
    ],
)
def _stage2(xm_hbm, s_hbm, out_hbm, idx_v, val_v, out_v, fold_v, sem):
    wid = lax.axis_index("s") * NC + lax.axis_index("c")
    row0 = wid * RPW
    lane = lax.iota(jnp.int32, L)
    tail_mask = lane >= jnp.int32(8)
    zeros = jnp.zeros((L,), jnp.float32)

    def chunk_body(ci, carry):
        base = (row0 + ci * C) * LX
        pltpu.sync_copy(xm_hbm.at[pl.ds(base, CW)], idx_v)
        pltpu.async_copy(s_hbm.at[idx_v], val_v, sem).wait()

        def grp_body(g, carry2):
            def row_body(r, acc):
                rbase = (g * L + r) * LX
                part = val_v[pl.ds(rbase, L)]
                for k in range(1, 12):
                    part = part + val_v[pl.ds(rbase + k * L, L)]
                tail = val_v[pl.ds(rbase + 184, L)]
                part = part + jnp.where(tail_mask, tail, zeros)
                for sh in (8, 4, 2, 1):
                    fold_v[pl.ds(0, L)] = part
                    part = part + fold_v[pl.ds(sh, L)]
                fold_v[pl.ds(r, L)] = part
                shifted = fold_v[pl.ds(0, L)]
                return jnp.where(lane == r, shifted, acc)

            acc = lax.fori_loop(0, L, row_body, zeros, unroll=2)
            y = jnp.float32(1.0) / (jnp.float32(1.0) + jnp.exp(-acc))
            y = y * jnp.float32(1e4)
            y = (y + jnp.float32(_ROUND_MAGIC)) - jnp.float32(_ROUND_MAGIC)
            y = y * jnp.float32(1e-4)
            out_v[pl.ds(ci * C + g * L, L)] = y
            return carry2

        lax.fori_loop(0, C // L, grp_body, 0)
        return carry

    lax.fori_loop(0, NCHUNK, chunk_body, 0)
    pltpu.sync_copy(out_v, out_hbm.at[pl.ds(row0, RPW)])


def kernel(x, emb, W, b):
    w = W.reshape(D).astype(jnp.float32)
    wd = jnp.kron(jnp.eye(PACK, dtype=jnp.float32), (w / LX)[:, None])
    bscal = (b.astype(jnp.float32) / LX).reshape(1, 1)
    s = _stage1(emb.reshape(G, 128), wd, bscal).reshape(VP)
    xi = x.astype(jnp.int32)
    xm = ((xi % PACK) * NCOLP + xi // PACK).reshape(B * LX)
    return _stage2(xm, s).reshape(B, 1)


# speedup vs baseline: 26.1141x; 3.1789x over previous
import functools

import jax
import jax.numpy as jnp
from jax import lax
from jax.experimental import pallas as pl
from jax.experimental.pallas import tpu as pltpu
from jax.experimental.pallas import tpu_sc as plsc

B = 16384
LX = 200
V = 1_000_000
D = 16

RBLK = 8192
NROWP = 123 * RBLK

NC, NS, L = 2, 16, 16
NW = NC * NS
RPW = B // NW
C = 256
NCHUNK = RPW // C
F = 8

_ROUND_MAGIC = 8388608.0


def _stage1_body(e_ref, w_ref, b_ref, o_ref):
    o_ref[...] = (
        jnp.sum(e_ref[...] * w_ref[...], axis=0, keepdims=True) + b_ref[0, 0]
    )


def _stage1(embT, w1, bscal):
    return pl.pallas_call(
        _stage1_body,
        grid=(NROWP // RBLK,),
        in_specs=[
            pl.BlockSpec((D, RBLK), lambda i: (0, i)),
            pl.BlockSpec((D, 1), lambda i: (0, 0)),
            pl.BlockSpec(memory_space=pltpu.SMEM),
        ],
        out_specs=pl.BlockSpec((1, RBLK), lambda i: (0, i)),
        out_shape=jax.ShapeDtypeStruct((1, NROWP), jnp.float32),
    )(embT, w1, bscal)


_MESH = plsc.VectorSubcoreMesh(core_axis_name="c", subcore_axis_name="s")


@functools.partial(
    pl.kernel,
    mesh=_MESH,
    out_type=jax.ShapeDtypeStruct((B,), jnp.float32),
    scratch_types=[
        pltpu.VMEM((LX * C,), jnp.int32),
        pltpu.VMEM((LX * C,), jnp.float32),
        pltpu.VMEM((RPW,), jnp.float32),
        pltpu.SemaphoreType.DMA((F,)),
        pltpu.SemaphoreType.DMA((F,)),
    ],
)
def _stage2(xt_hbm, s_hbm, out_hbm, idx_v, val_v, out_v, semA, semB):
    wid = lax.axis_index("s") * NC + lax.axis_index("c")
    col0w = wid * RPW
    nacc = C // L

    def idx_row(l):
        return idx_v.at[pl.ds(l * C, C)]

    def val_row(l):
        return val_v.at[pl.ds(l * C, C)]

    def chunk_body(ci, carry):
        col0 = col0w + ci * C
        for j in range(F):
            pltpu.async_copy(
                xt_hbm.at[j, pl.ds(col0, C)], idx_row(j), semA.at[j]
            )
        for j in range(F):
            pltpu.make_async_copy(
                xt_hbm.at[j, pl.ds(col0, C)], idx_row(j), semA.at[j]
            ).wait()
            pltpu.async_copy(s_hbm.at[idx_row(j)], val_row(j), semB.at[j])
            pltpu.async_copy(
                xt_hbm.at[j + F, pl.ds(col0, C)], idx_row(j + F), semA.at[j]
            )

        def grp_body(k, accs):
            new = list(accs)
            for j in range(F):
                l = k * F + j
                pltpu.make_async_copy(
                    s_hbm.at[idx_row(l)], val_row(l), semB.at[j]
                ).wait()

                @pl.when(k < LX // F - 1)
                def _():
                    pltpu.make_async_copy(
                        xt_hbm.at[j, pl.ds(col0, C)],
                        idx_row(l + F),
                        semA.at[j],
                    ).wait()
                    pltpu.async_copy(
                        s_hbm.at[idx_row(l + F)], val_row(l + F), semB.at[j]
                    )

                @pl.when(k < LX // F - 2)
                def _():
                    pltpu.async_copy(
                        xt_hbm.at[l + 2 * F, pl.ds(col0, C)],
                        idx_row(l + 2 * F),
                        semA.at[j],
                    )

                for a in range(nacc):
                    new[a] = new[a] + val_v[pl.ds(l * C + a * L, L)]
            return tuple(new)

        accs = lax.fori_loop(
            0,
            LX // F,
            grp_body,
            tuple(jnp.zeros((L,), jnp.float32) for _ in range(nacc)),
        )
        for a in range(nacc):
            y = jnp.float32(1.0) / (jnp.float32(1.0) + jnp.exp(-accs[a]))
            y = y * jnp.float32(1e4)
            y = (y + jnp.float32(_ROUND_MAGIC)) - jnp.float32(_ROUND_MAGIC)
            y = y * jnp.float32(1e-4)
            out_v[pl.ds(ci * C + a * L, L)] = y
        return carry

    lax.fori_loop(0, NCHUNK, chunk_body, 0)
    pltpu.sync_copy(out_v, out_hbm.at[pl.ds(col0w, RPW)])


def kernel(x, emb, W, b):
    w1 = (W.astype(jnp.float32) / LX).reshape(1, D).T
    bscal = (b.astype(jnp.float32) / LX).reshape(1, 1)
    s = _stage1(emb.T, w1, bscal).reshape(NROWP)
    xt = x.astype(jnp.int32).T
    return _stage2(xt, s).reshape(B, 1)


# speedup vs baseline: 32.5172x; 1.2452x over previous
import functools

import jax
import jax.numpy as jnp
from jax import lax
from jax.experimental import pallas as pl
from jax.experimental.pallas import tpu as pltpu
from jax.experimental.pallas import tpu_sc as plsc

B = 16384
LX = 200
V = 1_000_000
D = 16

RBLK = 32768
NROWP = 31 * RBLK

NC, NS, L = 2, 16, 16
NW = NC * NS
RPW = B // NW
C = 256
NCHUNK = RPW // C
F = 8

_ROUND_MAGIC = 8388608.0


def _stage1_body(e_ref, w_ref, b_ref, o_ref):
    o_ref[...] = (
        jnp.sum(e_ref[...] * w_ref[...], axis=0, keepdims=True) + b_ref[0, 0]
    )


def _stage1(embT, w1, bscal):
    return pl.pallas_call(
        _stage1_body,
        grid=(NROWP // RBLK,),

        in_specs=[
            pl.BlockSpec((D, RBLK), lambda i: (0, i)),
            pl.BlockSpec((D, 1), lambda i: (0, 0)),
            pl.BlockSpec(memory_space=pltpu.SMEM),
        ],
        out_specs=pl.BlockSpec((1, RBLK), lambda i: (0, i)),
        out_shape=jax.ShapeDtypeStruct((1, NROWP), jnp.float32),
    )(embT, w1, bscal)


_MESH = plsc.VectorSubcoreMesh(core_axis_name="c", subcore_axis_name="s")


@functools.partial(
    pl.kernel,
    mesh=_MESH,
    out_type=jax.ShapeDtypeStruct((B,), jnp.float32),
    scratch_types=[
        pltpu.VMEM((LX * C,), jnp.int32),
        pltpu.VMEM((LX * C,), jnp.float32),
        pltpu.VMEM((RPW,), jnp.float32),
        pltpu.SemaphoreType.DMA((F,)),
        pltpu.SemaphoreType.DMA((F,)),
    ],
)
def _stage2(xt_hbm, s_hbm, out_hbm, idx_v, val_v, out_v, semA, semB):
    wid = lax.axis_index("s") * NC + lax.axis_index("c")
    col0w = wid * RPW
    nacc = C // L

    def idx_row(l):
        return idx_v.at[pl.ds(l * C, C)]

    def val_row(l):
        return val_v.at[pl.ds(l * C, C)]

    def chunk_body(ci, carry):
        col0 = col0w + ci * C
        for j in range(F):
            pltpu.async_copy(
                xt_hbm.at[j, pl.ds(col0, C)], idx_row(j), semA.at[j]
            )
        for j in range(F):
            pltpu.make_async_copy(
                xt_hbm.at[j, pl.ds(col0, C)], idx_row(j), semA.at[j]
            ).wait()
            pltpu.async_copy(s_hbm.at[idx_row(j)], val_row(j), semB.at[j])
            pltpu.async_copy(
                xt_hbm.at[j + F, pl.ds(col0, C)], idx_row(j + F), semA.at[j]
            )

        def grp_body(k, accs):
            new = list(accs)
            for j in range(F):
                l = k * F + j
                pltpu.make_async_copy(
                    s_hbm.at[idx_row(l)], val_row(l), semB.at[j]
                ).wait()

                @pl.when(k < LX // F - 1)
                def _():
                    pltpu.make_async_copy(
                        xt_hbm.at[j, pl.ds(col0, C)],
                        idx_row(l + F),
                        semA.at[j],
                    ).wait()
                    pltpu.async_copy(
                        s_hbm.at[idx_row(l + F)], val_row(l + F), semB.at[j]
                    )

                @pl.when(k < LX // F - 2)
                def _():
                    pltpu.async_copy(
                        xt_hbm.at[l + 2 * F, pl.ds(col0, C)],
                        idx_row(l + 2 * F),
                        semA.at[j],
                    )

                for a in range(nacc):
                    new[a] = new[a] + val_v[pl.ds(l * C + a * L, L)]
            return tuple(new)

        accs = lax.fori_loop(
            0,
            LX // F,
            grp_body,
            tuple(jnp.zeros((L,), jnp.float32) for _ in range(nacc)),
        )
        for a in range(nacc):
            y = jnp.float32(1.0) / (jnp.float32(1.0) + jnp.exp(-accs[a]))
            y = y * jnp.float32(1e4)
            y = (y + jnp.float32(_ROUND_MAGIC)) - jnp.float32(_ROUND_MAGIC)
            y = y * jnp.float32(1e-4)
            out_v[pl.ds(ci * C + a * L, L)] = y
        return carry

    lax.fori_loop(0, NCHUNK, chunk_body, 0)
    pltpu.sync_copy(out_v, out_hbm.at[pl.ds(col0w, RPW)])


def kernel(x, emb, W, b):
    w1 = (W.astype(jnp.float32) / LX).reshape(1, D).T
    bscal = (b.astype(jnp.float32) / LX).reshape(1, 1)
    s = _stage1(emb.T, w1, bscal).reshape(NROWP)
    xt = x.astype(jnp.int32).T
    return _stage2(xt, s).reshape(B, 1)
